# speedup vs baseline: 1.7543x; 1.7543x over previous
import functools

import jax
import jax.numpy as jnp
from jax import lax
from jax.experimental import pallas as pl
from jax.experimental.pallas import tpu as pltpu
from jax.experimental.pallas import tpu_sc as plsc

BATCH = 16384
DIM = 128

_info = plsc.get_sparse_core_info()
_NC = _info.num_cores
_NS = _info.num_subcores
_NW = _NC * _NS
_BPW = BATCH // _NW
_CHUNK = 128
_NCH = _BPW // _CHUNK

_mesh = plsc.VectorSubcoreMesh(core_axis_name="c", subcore_axis_name="s")


@functools.partial(
    pl.kernel,
    mesh=_mesh,
    out_type=[
        jax.ShapeDtypeStruct((BATCH, DIM), jnp.float32),
        jax.ShapeDtypeStruct((BATCH, DIM), jnp.float32),
        jax.ShapeDtypeStruct((BATCH,), jnp.int32),
    ],
    scratch_types=[
        pltpu.VMEM((_BPW,), jnp.int32),
        pltpu.VMEM((_CHUNK, DIM), jnp.float32),
        pltpu.VMEM((_CHUNK, DIM), jnp.float32),
        pltpu.VMEM((_CHUNK, DIM), jnp.float32),
        pltpu.VMEM((_CHUNK, DIM), jnp.float32),
        pltpu.VMEM((_BPW,), jnp.int32),
        pltpu.SemaphoreType.DMA,
        pltpu.SemaphoreType.DMA,
        pltpu.SemaphoreType.DMA,
        pltpu.SemaphoreType.DMA,
        pltpu.SemaphoreType.DMA,
    ],
)
def _gather3(n_id_hbm, memory_hbm, pos_memory_hbm, last_update_hbm,
             z_hbm, pos_z_hbm, lu_hbm,
             idx_v, buf_m0, buf_m1, buf_p0, buf_p1, lu_v,
             sem_gm, sem_gp, sem_om, sem_op, sem_lu):
    wid = lax.axis_index("s") * _NC + lax.axis_index("c")
    base = wid * _BPW

    buf_m = (buf_m0, buf_m1)
    buf_p = (buf_p0, buf_p1)

    pltpu.sync_copy(n_id_hbm.at[pl.ds(base, _BPW)], idx_v)

    def idx_slice(ci):
        return idx_v.at[pl.ds(ci * _CHUNK, _CHUNK)]

    lu_copies = []
    for ci in range(_NCH):
        lu_copies.append(pltpu.async_copy(
            last_update_hbm.at[idx_slice(ci)],
            lu_v.at[pl.ds(ci * _CHUNK, _CHUNK)], sem_lu))

    g_m = [None] * _NCH
    g_p = [None] * _NCH
    o_m = [None] * _NCH
    o_p = [None] * _NCH
    for ci in range(min(2, _NCH)):
        g_m[ci] = pltpu.async_copy(
            memory_hbm.at[idx_slice(ci)], buf_m[ci % 2], sem_gm)
        g_p[ci] = pltpu.async_copy(
            pos_memory_hbm.at[idx_slice(ci)], buf_p[ci % 2], sem_gp)

    for ci in range(_NCH):
        out_rows = pl.ds(base + ci * _CHUNK, _CHUNK)
        g_m[ci].wait()
        o_m[ci] = pltpu.async_copy(buf_m[ci % 2], z_hbm.at[out_rows], sem_om)
        g_p[ci].wait()
        o_p[ci] = pltpu.async_copy(buf_p[ci % 2], pos_z_hbm.at[out_rows], sem_op)
        nxt = ci + 2
        if nxt < _NCH:
            o_m[ci].wait()
            o_m[ci] = None
            g_m[nxt] = pltpu.async_copy(
                memory_hbm.at[idx_slice(nxt)], buf_m[nxt % 2], sem_gm)
            o_p[ci].wait()
            o_p[ci] = None
            g_p[nxt] = pltpu.async_copy(
                pos_memory_hbm.at[idx_slice(nxt)], buf_p[nxt % 2], sem_gp)

    for ci in range(_NCH):
        if o_m[ci] is not None:
            o_m[ci].wait()
        if o_p[ci] is not None:
            o_p[ci].wait()
    for c in lu_copies:
        c.wait()
    pltpu.sync_copy(lu_v, lu_hbm.at[pl.ds(base, _BPW)])


def kernel(n_id, memory, pos_memory, last_update):
    z, pos_z, lu = _gather3(n_id, memory, pos_memory, last_update)
    return (z, pos_z, lu)
